# baseline (speedup 1.0000x reference)
import functools

import jax
import jax.numpy as jnp
from jax import lax
from jax.experimental import pallas as pl
from jax.experimental.pallas import tpu as pltpu
from jax.experimental.pallas import tpu_sc as plsc

NC = 2
NS = 16
NW = NC * NS
BLK = 128
SCALE = 8.0
L = 16


def _embed_sc(table2, X4, B, H, D):
    blks_per_w = B // (NW * BLK)
    H8 = X4.shape[0]

    mesh = plsc.VectorSubcoreMesh(core_axis_name="c", subcore_axis_name="s")

    @functools.partial(
        pl.kernel,
        mesh=mesh,
        compiler_params=pltpu.CompilerParams(
            use_tc_tiling_on_sc=False, needs_layout_passes=False),
        out_type=jax.ShapeDtypeStruct((H, D // 8, B // BLK, 8, BLK), jnp.float32),
        scratch_types=[
            pltpu.VMEM((H8, 1, 8, BLK), jnp.int32),
            pltpu.VMEM((BLK, D), jnp.float32),
            pltpu.VMEM((BLK, D), jnp.float32),
            pltpu.VMEM((D // 8, 1, 8, BLK), jnp.float32),
            pltpu.SemaphoreType.DMA,
            pltpu.SemaphoreType.DMA,
        ],
    )
    def k(tab_hbm, idx_hbm, out_hbm, idxq, buf0, buf1, obuf, sem0, sem1):
        wid = lax.axis_index("s") * NC + lax.axis_index("c")

        def double_idx(_):
            def row(h, c):
                ch = lax.div(h, 8)
                cl = lax.rem(h, 8)
                for kk in range(BLK // L):
                    sl = pl.ds(kk * L, L)
                    idxq[ch, 0, cl, sl] = idxq[ch, 0, cl, sl] * 2
                return c

            lax.fori_loop(0, H, row, 0)

        def fire(h, buf, sem):
            ch = lax.div(h, 8)
            cl = lax.rem(h, 8)
            pltpu.async_copy(tab_hbm.at[idxq.at[ch, 0, cl]], buf, sem)

        def drain(h, buf, sem):
            ch = lax.div(h, 8)
            cl = lax.rem(h, 8)
            pltpu.make_async_copy(tab_hbm.at[idxq.at[ch, 0, cl]], buf, sem).wait()

        def transpose_store(h, gblk, buf):
            def col(d, c):
                dh = lax.div(d, 8)
                dl = lax.rem(d, 8)
                for kk in range(BLK // L):
                    rows = kk * L + lax.iota(jnp.int32, L)
                    cols = jnp.full((L,), d, jnp.int32)
                    v = plsc.load_gather(buf, [rows, cols])
                    obuf[dh, 0, dl, pl.ds(kk * L, L)] = v * SCALE
                return c

            lax.fori_loop(0, D, col, 0)
            pltpu.sync_copy(obuf, out_hbm.at[h, pl.ds(0, D // 8), pl.ds(gblk, 1)])

        def blk_body(blk, c):
            gblk = wid * blks_per_w + blk
            pltpu.sync_copy(idx_hbm.at[pl.ds(0, H8), pl.ds(gblk, 1)], idxq)
            double_idx(blk)
            fire(0, buf0, sem0)

            def h_body(h, cc):
                for b in range(2):
                    buf = (buf0, buf1)[b]
                    sem = (sem0, sem1)[b]
                    nbuf = (buf0, buf1)[1 - b]
                    nsem = (sem0, sem1)[1 - b]

                    @pl.when(lax.rem(h, 2) == b)
                    def _():
                        @pl.when(h + 1 < H)
                        def _():
                            fire(h + 1, nbuf, nsem)

                        drain(h, buf, sem)
                        transpose_store(h, gblk, buf)

                return cc

            lax.fori_loop(0, H, h_body, 0)
            return c

        lax.fori_loop(0, blks_per_w, blk_body, 0)

    return k(table2, X4)


def kernel(X, embedding_matrix):
    B, H = X.shape
    V, D = embedding_matrix.shape
    H8 = (H + 7) // 8 * 8
    t2 = jnp.pad(embedding_matrix, ((0, 0), (0, 128 - D))).reshape(2 * V, D)
    x4 = (
        jnp.pad(X.astype(jnp.int32), ((0, 0), (0, H8 - H)))
        .reshape(B // BLK, BLK, H8 // 8, 8)
        .transpose(2, 0, 3, 1)
    )
    out5d = _embed_sc(t2, x4, B, H, D)
    return out5d.transpose(2, 4, 0, 1, 3).reshape(B, H, D)


# speedup vs baseline: 1.0246x; 1.0246x over previous
import functools

import jax
import jax.numpy as jnp
from jax import lax
from jax.experimental import pallas as pl
from jax.experimental.pallas import tpu as pltpu
from jax.experimental.pallas import tpu_sc as plsc

NC = 2
NS = 16
NW = NC * NS
BLK = 128
SCALE = 8.0
L = 16


def _embed_sc(table2, X4, B, H, D):
    blks_per_w = B // (NW * BLK)
    H8 = X4.shape[0]

    mesh = plsc.VectorSubcoreMesh(core_axis_name="c", subcore_axis_name="s")

    @functools.partial(
        pl.kernel,
        mesh=mesh,
        compiler_params=pltpu.CompilerParams(
            use_tc_tiling_on_sc=False, needs_layout_passes=False),
        out_type=jax.ShapeDtypeStruct((H, D // 8, B // BLK, 8, BLK), jnp.float32),
        scratch_types=[
            pltpu.VMEM((H8, 1, 8, BLK), jnp.int32),
            pltpu.VMEM((BLK, D), jnp.float32),
            pltpu.VMEM((BLK, D), jnp.float32),
            pltpu.VMEM((D // 8, 1, 8, BLK), jnp.float32),
            pltpu.VMEM((D // 8, 1, 8, BLK), jnp.float32),
            pltpu.SemaphoreType.DMA,
            pltpu.SemaphoreType.DMA,
            pltpu.SemaphoreType.DMA,
            pltpu.SemaphoreType.DMA,
        ],
    )
    def k(tab_hbm, idx_hbm, out_hbm, idxq, buf0, buf1, ob0, ob1,
          sem0, sem1, osem0, osem1):
        wid = lax.axis_index("s") * NC + lax.axis_index("c")
        rowsk = [kk * L + lax.iota(jnp.int32, L) for kk in range(BLK // L)]

        def double_idx(_):
            def row(h, c):
                ch = lax.div(h, 8)
                cl = lax.rem(h, 8)
                for kk in range(BLK // L):
                    sl = pl.ds(kk * L, L)
                    idxq[ch, 0, cl, sl] = idxq[ch, 0, cl, sl] * 2
                return c

            lax.fori_loop(0, H, row, 0)

        def fire(h, buf, sem):
            ch = lax.div(h, 8)
            cl = lax.rem(h, 8)
            pltpu.async_copy(tab_hbm.at[idxq.at[ch, 0, cl]], buf, sem)

        def drain(h, buf, sem):
            ch = lax.div(h, 8)
            cl = lax.rem(h, 8)
            pltpu.make_async_copy(tab_hbm.at[idxq.at[ch, 0, cl]], buf, sem).wait()

        def out_dst(h, gblk):
            return out_hbm.at[h, pl.ds(0, D // 8), pl.ds(gblk, 1)]

        def transpose(buf, obuf):
            def grp(dh, c):
                dbase = jnp.full((L,), dh * 8, jnp.int32)
                for dl in range(8):
                    cols = dbase + dl
                    for kk in range(BLK // L):
                        v = plsc.load_gather(buf, [rowsk[kk], cols])
                        obuf[dh, 0, dl, pl.ds(kk * L, L)] = v * SCALE
                return c

            lax.fori_loop(0, D // 8, grp, 0)

        def blk_body(blk, c):
            gblk = wid * blks_per_w + blk
            pltpu.sync_copy(idx_hbm.at[pl.ds(0, H8), pl.ds(gblk, 1)], idxq)
            double_idx(blk)
            fire(0, buf0, sem0)

            def h_body(h, cc):
                for b in range(2):
                    buf = (buf0, buf1)[b]
                    sem = (sem0, sem1)[b]
                    nbuf = (buf0, buf1)[1 - b]
                    nsem = (sem0, sem1)[1 - b]
                    obuf = (ob0, ob1)[b]
                    osem = (osem0, osem1)[b]

                    @pl.when(lax.rem(h, 2) == b)
                    def _():
                        @pl.when(h + 1 < H)
                        def _():
                            fire(h + 1, nbuf, nsem)

                        @pl.when(h >= 2)
                        def _():
                            pltpu.make_async_copy(
                                obuf, out_dst(h - 2, gblk), osem).wait()

                        drain(h, buf, sem)
                        transpose(buf, obuf)
                        pltpu.async_copy(obuf, out_dst(h, gblk), osem)

                return cc

            lax.fori_loop(0, H, h_body, 0)
            pltpu.make_async_copy(ob0, out_dst(H - 2, gblk), osem0).wait()
            pltpu.make_async_copy(ob1, out_dst(H - 1, gblk), osem1).wait()
            return c

        lax.fori_loop(0, blks_per_w, blk_body, 0)

    return k(table2, X4)


def kernel(X, embedding_matrix):
    B, H = X.shape
    V, D = embedding_matrix.shape
    H8 = (H + 7) // 8 * 8
    t2 = jnp.pad(embedding_matrix, ((0, 0), (0, 128 - D))).reshape(2 * V, D)
    x4 = (
        jnp.pad(X.astype(jnp.int32), ((0, 0), (0, H8 - H)))
        .reshape(B // BLK, BLK, H8 // 8, 8)
        .transpose(2, 0, 3, 1)
    )
    out5d = _embed_sc(t2, x4, B, H, D)
    return out5d.transpose(2, 4, 0, 1, 3).reshape(B, H, D)


# speedup vs baseline: 1.3664x; 1.3336x over previous
import functools

import jax
import jax.numpy as jnp
from jax import lax
from jax.experimental import pallas as pl
from jax.experimental.pallas import tpu as pltpu
from jax.experimental.pallas import tpu_sc as plsc

NC = 2
NS = 16
NW = NC * NS
BLK = 128
SCALE = 8.0
L = 16


def _embed_sc(table2, X4, B, H, D):
    blks_per_w = B // (NW * BLK)
    H8 = X4.shape[0]

    mesh = plsc.VectorSubcoreMesh(core_axis_name="c", subcore_axis_name="s")

    @functools.partial(
        pl.kernel,
        mesh=mesh,
        compiler_params=pltpu.CompilerParams(
            use_tc_tiling_on_sc=False, needs_layout_passes=False),
        out_type=jax.ShapeDtypeStruct((H, D // 8, B // BLK, 8, BLK), jnp.float32),
        scratch_types=[
            pltpu.VMEM((H8, 1, 8, BLK), jnp.int32),
            pltpu.VMEM((BLK, D), jnp.float32),
            pltpu.VMEM((BLK, D), jnp.float32),
            pltpu.VMEM((D // 8, 1, 8, BLK), jnp.float32),
            pltpu.VMEM((D // 8, 1, 8, BLK), jnp.float32),
            pltpu.SemaphoreType.DMA,
            pltpu.SemaphoreType.DMA,
            pltpu.SemaphoreType.DMA,
            pltpu.SemaphoreType.DMA,
        ],
    )
    def k(tab_hbm, idx_hbm, out_hbm, idxq, buf0, buf1, ob0, ob1,
          sem0, sem1, osem0, osem1):
        wid = lax.axis_index("s") * NC + lax.axis_index("c")
        rowsk = [kk * L + lax.iota(jnp.int32, L) for kk in range(BLK // L)]

        def double_idx(_):
            def row(h, c):
                ch = lax.div(h, 8)
                cl = lax.rem(h, 8)
                for kk in range(BLK // L):
                    sl = pl.ds(kk * L, L)
                    idxq[ch, 0, cl, sl] = idxq[ch, 0, cl, sl] * 2
                return c

            lax.fori_loop(0, H, row, 0)

        def fire(h, buf, sem):
            ch = lax.div(h, 8)
            cl = lax.rem(h, 8)
            pltpu.async_copy(tab_hbm.at[idxq.at[ch, 0, cl]], buf, sem)

        def drain(h, buf, sem):
            ch = lax.div(h, 8)
            cl = lax.rem(h, 8)
            pltpu.make_async_copy(tab_hbm.at[idxq.at[ch, 0, cl]], buf, sem).wait()

        def out_dst(h, gblk):
            return out_hbm.at[h, pl.ds(0, D // 8), pl.ds(gblk, 1)]

        def transpose(buf, obuf):
            @plsc.parallel_loop(0, D, 1, unroll=2)
            def col(d):
                dh = lax.div(d, 8)
                dl = lax.rem(d, 8)
                cols = jnp.full((L,), d, jnp.int32)
                vs = [
                    plsc.load_gather(buf, [rowsk[kk], cols]) * SCALE
                    for kk in range(BLK // L)
                ]
                for kk in range(BLK // L):
                    obuf[dh, 0, dl, pl.ds(kk * L, L)] = vs[kk]

        def blk_body(blk, c):
            gblk = wid * blks_per_w + blk
            pltpu.sync_copy(idx_hbm.at[pl.ds(0, H8), pl.ds(gblk, 1)], idxq)
            double_idx(blk)
            fire(0, buf0, sem0)

            def h_body(h, cc):
                for b in range(2):
                    buf = (buf0, buf1)[b]
                    sem = (sem0, sem1)[b]
                    nbuf = (buf0, buf1)[1 - b]
                    nsem = (sem0, sem1)[1 - b]
                    obuf = (ob0, ob1)[b]
                    osem = (osem0, osem1)[b]

                    @pl.when(lax.rem(h, 2) == b)
                    def _():
                        @pl.when(h + 1 < H)
                        def _():
                            fire(h + 1, nbuf, nsem)

                        @pl.when(h >= 2)
                        def _():
                            pltpu.make_async_copy(
                                obuf, out_dst(h - 2, gblk), osem).wait()

                        drain(h, buf, sem)
                        transpose(buf, obuf)
                        pltpu.async_copy(obuf, out_dst(h, gblk), osem)

                return cc

            lax.fori_loop(0, H, h_body, 0)
            pltpu.make_async_copy(ob0, out_dst(H - 2, gblk), osem0).wait()
            pltpu.make_async_copy(ob1, out_dst(H - 1, gblk), osem1).wait()
            return c

        lax.fori_loop(0, blks_per_w, blk_body, 0)

    return k(table2, X4)


def kernel(X, embedding_matrix):
    B, H = X.shape
    V, D = embedding_matrix.shape
    H8 = (H + 7) // 8 * 8
    t2 = jnp.pad(embedding_matrix, ((0, 0), (0, 128 - D))).reshape(2 * V, D)
    x4 = (
        jnp.pad(X.astype(jnp.int32), ((0, 0), (0, H8 - H)))
        .reshape(B // BLK, BLK, H8 // 8, 8)
        .transpose(2, 0, 3, 1)
    )
    out5d = _embed_sc(t2, x4, B, H, D)
    return out5d.transpose(2, 4, 0, 1, 3).reshape(B, H, D)


# speedup vs baseline: 2.4873x; 1.8204x over previous
import functools

import jax
import jax.numpy as jnp
from jax import lax
from jax.experimental import pallas as pl
from jax.experimental.pallas import tpu as pltpu
from jax.experimental.pallas import tpu_sc as plsc

NC = 2
NS = 16
NW = NC * NS
BLK = 128
SCALE = 8.0
L = 16


def _embed_sc(table2, X4, B, H, D):
    blks_per_w = B // (NW * BLK)
    H8 = X4.shape[0]

    mesh = plsc.VectorSubcoreMesh(core_axis_name="c", subcore_axis_name="s")

    @functools.partial(
        pl.kernel,
        mesh=mesh,
        compiler_params=pltpu.CompilerParams(
            use_tc_tiling_on_sc=False, needs_layout_passes=False),
        out_type=jax.ShapeDtypeStruct((H, D // 8, B // BLK, 8, BLK), jnp.float32),
        scratch_types=[
            pltpu.VMEM((H8, 1, 8, BLK), jnp.int32),
            pltpu.VMEM((BLK, D), jnp.float32),
            pltpu.VMEM((BLK, D), jnp.float32),
            pltpu.VMEM((D // 8, 1, 8, BLK + 1), jnp.float32),
            pltpu.VMEM((D // 8, 1, 8, BLK + 1), jnp.float32),
            pltpu.SemaphoreType.DMA,
            pltpu.SemaphoreType.DMA,
            pltpu.SemaphoreType.DMA,
            pltpu.SemaphoreType.DMA,
        ],
    )
    def k(tab_hbm, idx_hbm, out_hbm, idxq, buf0, buf1, ob0, ob1,
          sem0, sem1, osem0, osem1):
        wid = lax.axis_index("s") * NC + lax.axis_index("c")
        iota = lax.iota(jnp.int32, L)
        dhq = [(q * L + iota) // 8 for q in range(D // L)]
        dlq = [(q * L + iota) % 8 for q in range(D // L)]
        zeroq = jnp.zeros((L,), jnp.int32)

        def double_idx(_):
            def row(h, c):
                ch = lax.div(h, 8)
                cl = lax.rem(h, 8)
                for kk in range(BLK // L):
                    sl = pl.ds(kk * L, L)
                    idxq[ch, 0, cl, sl] = idxq[ch, 0, cl, sl] * 2
                return c

            lax.fori_loop(0, H, row, 0)

        def fire(h, buf, sem):
            ch = lax.div(h, 8)
            cl = lax.rem(h, 8)
            pltpu.async_copy(tab_hbm.at[idxq.at[ch, 0, cl]], buf, sem)

        def drain(h, buf, sem):
            ch = lax.div(h, 8)
            cl = lax.rem(h, 8)
            pltpu.make_async_copy(tab_hbm.at[idxq.at[ch, 0, cl]], buf, sem).wait()

        def out_dst(h, gblk):
            return out_hbm.at[h, pl.ds(0, D // 8), pl.ds(gblk, 1)]

        def out_src(obuf):
            return obuf.at[pl.ds(0, D // 8), pl.ds(0, 1), pl.ds(0, 8),
                           pl.ds(0, BLK)]

        def transpose(buf, obuf):
            @plsc.parallel_loop(0, BLK, 1, unroll=2)
            def rowb(b):
                bvec = jnp.full((L,), b, jnp.int32)
                vs = [buf[b, pl.ds(q * L, L)] * SCALE for q in range(D // L)]
                for q in range(D // L):
                    plsc.store_scatter(obuf, [dhq[q], zeroq, dlq[q], bvec],
                                       vs[q])

        def blk_body(blk, c):
            gblk = wid * blks_per_w + blk
            pltpu.sync_copy(idx_hbm.at[pl.ds(0, H8), pl.ds(gblk, 1)], idxq)
            double_idx(blk)
            fire(0, buf0, sem0)

            def h_body(h, cc):
                for b in range(2):
                    buf = (buf0, buf1)[b]
                    sem = (sem0, sem1)[b]
                    nbuf = (buf0, buf1)[1 - b]
                    nsem = (sem0, sem1)[1 - b]
                    obuf = (ob0, ob1)[b]
                    osem = (osem0, osem1)[b]

                    @pl.when(lax.rem(h, 2) == b)
                    def _():
                        @pl.when(h + 1 < H)
                        def _():
                            fire(h + 1, nbuf, nsem)

                        @pl.when(h >= 2)
                        def _():
                            pltpu.make_async_copy(
                                out_src(obuf), out_dst(h - 2, gblk),
                                osem).wait()

                        drain(h, buf, sem)
                        transpose(buf, obuf)
                        pltpu.async_copy(out_src(obuf), out_dst(h, gblk),
                                         osem)

                return cc

            lax.fori_loop(0, H, h_body, 0)
            pltpu.make_async_copy(
                out_src(ob0), out_dst(H - 2, gblk), osem0).wait()
            pltpu.make_async_copy(
                out_src(ob1), out_dst(H - 1, gblk), osem1).wait()
            return c

        lax.fori_loop(0, blks_per_w, blk_body, 0)

    return k(table2, X4)


def kernel(X, embedding_matrix):
    B, H = X.shape
    V, D = embedding_matrix.shape
    H8 = (H + 7) // 8 * 8
    t2 = jnp.pad(embedding_matrix, ((0, 0), (0, 128 - D))).reshape(2 * V, D)
    x4 = (
        jnp.pad(X.astype(jnp.int32), ((0, 0), (0, H8 - H)))
        .reshape(B // BLK, BLK, H8 // 8, 8)
        .transpose(2, 0, 3, 1)
    )
    out5d = _embed_sc(t2, x4, B, H, D)
    return out5d.transpose(2, 4, 0, 1, 3).reshape(B, H, D)
